# baseline (speedup 1.0000x reference)
import jax
import jax.numpy as jnp
from jax.experimental import pallas as pl

_N = 2048
_D = 1024
_L = 64
_K = 8
_F = 2048
_COEFF = 0.01

_BN = 256
_GRID = _N // _BN


def _main_body(enc_ref, dec_ref, lab_ref, wenc_ref, benc_ref, w1_ref, w3_ref,
               w2_ref, out_ref):
    i = pl.program_id(0)

    logits = jnp.dot(enc_ref[...], wenc_ref[...],
                     preferred_element_type=jnp.float32) + benc_ref[...]

    iota_l = jax.lax.broadcasted_iota(jnp.int32, (_BN, _L), 1)
    lg = logits
    vals = []
    onehots = []
    for _ in range(_K):
        m = jnp.max(lg, axis=1, keepdims=True)
        sel = jnp.min(jnp.where(lg == m, iota_l, _L), axis=1, keepdims=True)
        oh = iota_l == sel
        vals.append(m)
        onehots.append(oh.astype(jnp.float32))
        lg = jnp.where(oh, -jnp.inf, lg)
    z = jnp.concatenate(vals, axis=1)

    czs = [z[:, 0:1]]
    for k in range(1, _K):
        czs.append(czs[-1] + z[:, k:k + 1])
    cz = jnp.concatenate(czs, axis=1)
    rk = jax.lax.broadcasted_iota(jnp.int32, (_BN, _K), 1).astype(
        jnp.float32) + 1.0
    cond = 1.0 + rk * z > cz
    ksel = jnp.sum(cond.astype(jnp.float32), axis=1, keepdims=True)
    czsel = jnp.sum(jnp.where(rk == ksel, cz, 0.0), axis=1, keepdims=True)
    tau = (czsel - 1.0) / ksel
    probs = jnp.maximum(z - tau, 0.0)

    p_safe = jnp.where(probs > 0, probs, 1.0)
    ent_sum = -jnp.sum(probs * jnp.log(p_safe))

    u = jnp.dot(dec_ref[...], w3_ref[...], preferred_element_type=jnp.float32)
    lab = lab_ref[...]
    acc = -_COEFF * ent_sum
    for k in range(_K):
        w1row = jnp.dot(onehots[k].astype(jnp.bfloat16), w1_ref[...],
                        preferred_element_type=jnp.float32)
        h = jnp.maximum(u + w1row, 0.0).astype(jnp.bfloat16)
        y = jnp.dot(h, w2_ref[...], preferred_element_type=jnp.float32)
        dlt = y - lab
        lc = jnp.sum(dlt * dlt, axis=1) * (1.0 / _D)
        acc = acc + jnp.sum(probs[:, k] * lc)

    acc2d = acc.reshape(1, 1)
    out_ref[...] = jnp.where(i == 0, acc2d, out_ref[...] + acc2d)


def kernel(encoder_input, decoder_input, labels, W_enc, b_enc, W1, W3, W2):
    out = pl.pallas_call(
        _main_body,
        grid=(_GRID,),
        in_specs=[
            pl.BlockSpec((_BN, _D), lambda i: (i, 0)),
            pl.BlockSpec((_BN, _D), lambda i: (i, 0)),
            pl.BlockSpec((_BN, _D), lambda i: (i, 0)),
            pl.BlockSpec((_D, _L), lambda i: (0, 0)),
            pl.BlockSpec((1, _L), lambda i: (0, 0)),
            pl.BlockSpec((_L, _F), lambda i: (0, 0)),
            pl.BlockSpec((_D, _F), lambda i: (0, 0)),
            pl.BlockSpec((_F, _D), lambda i: (0, 0)),
        ],
        out_specs=pl.BlockSpec((1, 1), lambda i: (0, 0)),
        out_shape=jax.ShapeDtypeStruct((1, 1), jnp.float32),
    )(encoder_input, decoder_input.astype(jnp.bfloat16), labels, W_enc,
      b_enc.reshape(1, _L), W1.astype(jnp.bfloat16),
      W3.astype(jnp.bfloat16), W2.astype(jnp.bfloat16))
    return out[0, 0] / _N


# speedup vs baseline: 1.1336x; 1.1336x over previous
import jax
import jax.numpy as jnp
from jax.experimental import pallas as pl

_N = 2048
_D = 1024
_L = 64
_K = 8
_F = 2048
_COEFF = 0.01

_BN = 512
_GRID = _N // _BN


def _main_body(enc_ref, dec_ref, lab_ref, wenc_ref, benc_ref, w1_ref, w3_ref,
               w2_ref, out_ref):
    i = pl.program_id(0)

    logits = jnp.dot(enc_ref[...], wenc_ref[...],
                     preferred_element_type=jnp.float32) + benc_ref[...]

    iota_l = jax.lax.broadcasted_iota(jnp.int32, (_BN, _L), 1)
    lg = logits
    vals = []
    onehots = []
    for _ in range(_K):
        m = jnp.max(lg, axis=1, keepdims=True)
        sel = jnp.min(jnp.where(lg == m, iota_l, _L), axis=1, keepdims=True)
        oh = iota_l == sel
        vals.append(m)
        onehots.append(oh.astype(jnp.float32))
        lg = jnp.where(oh, -jnp.inf, lg)
    z = jnp.concatenate(vals, axis=1)

    czs = [z[:, 0:1]]
    for k in range(1, _K):
        czs.append(czs[-1] + z[:, k:k + 1])
    cz = jnp.concatenate(czs, axis=1)
    rk = jax.lax.broadcasted_iota(jnp.int32, (_BN, _K), 1).astype(
        jnp.float32) + 1.0
    cond = 1.0 + rk * z > cz
    ksel = jnp.sum(cond.astype(jnp.float32), axis=1, keepdims=True)
    czsel = jnp.sum(jnp.where(rk == ksel, cz, 0.0), axis=1, keepdims=True)
    tau = (czsel - 1.0) / ksel
    probs = jnp.maximum(z - tau, 0.0)

    p_safe = jnp.where(probs > 0, probs, 1.0)
    ent_sum = -jnp.sum(probs * jnp.log(p_safe))

    u = jnp.dot(dec_ref[...], w3_ref[...], preferred_element_type=jnp.float32)
    lab = lab_ref[...]
    acc = -_COEFF * ent_sum
    for k in range(_K):
        w1row = jnp.dot(onehots[k], w1_ref[...],
                        preferred_element_type=jnp.float32)
        h = jnp.maximum(u + w1row, 0.0)
        y = jnp.dot(h, w2_ref[...], preferred_element_type=jnp.float32)
        dlt = y - lab
        lc = jnp.sum(dlt * dlt, axis=1) * (1.0 / _D)
        acc = acc + jnp.sum(probs[:, k] * lc)

    acc2d = acc.reshape(1, 1)
    out_ref[...] = jnp.where(i == 0, acc2d, out_ref[...] + acc2d)


def kernel(encoder_input, decoder_input, labels, W_enc, b_enc, W1, W3, W2):
    out = pl.pallas_call(
        _main_body,
        grid=(_GRID,),
        in_specs=[
            pl.BlockSpec((_BN, _D), lambda i: (i, 0)),
            pl.BlockSpec((_BN, _D), lambda i: (i, 0)),
            pl.BlockSpec((_BN, _D), lambda i: (i, 0)),
            pl.BlockSpec((_D, _L), lambda i: (0, 0)),
            pl.BlockSpec((1, _L), lambda i: (0, 0)),
            pl.BlockSpec((_L, _F), lambda i: (0, 0)),
            pl.BlockSpec((_D, _F), lambda i: (0, 0)),
            pl.BlockSpec((_F, _D), lambda i: (0, 0)),
        ],
        out_specs=pl.BlockSpec((1, 1), lambda i: (0, 0)),
        out_shape=jax.ShapeDtypeStruct((1, 1), jnp.float32),
    )(encoder_input, decoder_input, labels, W_enc,
      b_enc.reshape(1, _L), W1, W3, W2)
    return out[0, 0] / _N
